# baseline (speedup 1.0000x reference)
import jax
import jax.numpy as jnp
import numpy as np
from jax.experimental import pallas as pl
from jax.experimental.pallas import tpu as pltpu
import jax.experimental.pallas.tpu_sc as plsc

_N_HEAD = 16
_L = 16
_IMIN = np.int32(-(2 ** 31))
_IMAX = np.int32(2 ** 31 - 1)
_NCH = 2048 // _L
_NSC = 2048


def _proj_kernel(w_ref, ac_ref, emb_ref, a12_ref):
    h = jax.lax.dot_general(emb_ref[0].astype(jnp.bfloat16), w_ref[...],
                            (((1,), (1,)), ((), ())),
                            preferred_element_type=jnp.float32)
    a12_ref[0] = jax.lax.dot_general(h.astype(jnp.bfloat16), ac_ref[...],
                                     (((1,), (0,)), ((), ())),
                                     preferred_element_type=jnp.float32)


def _xsum(red_ref, v):
    for sh in (1, 2, 4, 8):
        red_ref[pl.ds(0, _L)] = v
        red_ref[pl.ds(_L, _L)] = v
        v = v + red_ref[pl.ds(sh, _L)]
    return v


def _xmax(red_ref, v):
    for sh in (1, 2, 4, 8):
        red_ref[pl.ds(0, _L)] = v
        red_ref[pl.ds(_L, _L)] = v
        v = jnp.maximum(v, red_ref[pl.ds(sh, _L)])
    return v


def _rank_sc_body(scal_ref, a2v_ref, out_ref, buf_ref, keys_ref, cm_ref,
                  red_ref, sv_ref, acc_ref, ls_ref, sem):
    c = jax.lax.axis_index("c")
    s = jax.lax.axis_index("s")
    b = c * 16 + s

    batches = a2v_ref.shape[0]
    @pl.when(b < batches)
    def _():
        cp = pltpu.make_async_copy(scal_ref, sv_ref, sem)
        cp.start()
        cp.wait()
        cp2 = pltpu.make_async_copy(a2v_ref.at[b], buf_ref, sem)
        cp2.start()
        cp2.wait()
        lane = jax.lax.iota(jnp.int32, _L)
        sv = sv_ref[pl.ds(0, _L)]
        ns = _xmax(red_ref, jnp.where(lane == b, sv, _IMIN))
        nt = _xmax(red_ref, jnp.where(lane == b + 4, sv, _IMIN))
        kk = (2 * ns * 52429) >> 18

        def key_body(i, carry):
            x = buf_ref[pl.ds(i * _L, _L)]
            bits = jax.lax.bitcast_convert_type(x, jnp.int32)
            keym = bits ^ ((bits >> 31) & jnp.int32(0x7FFFFFFF))
            idx = lane + i * _L
            keys_ref[pl.ds(i * _L, _L)] = jnp.where(idx < nt, keym, _IMIN)
            return carry

        jax.lax.fori_loop(0, _NCH, key_body, 0)

        def count(t):
            acc_ref[pl.ds(0, _L)] = jnp.zeros((_L,), jnp.int32)

            def cbody(i, carry):
                k = keys_ref[pl.ds(i * _L, _L)]
                acc_ref[pl.ds(0, _L)] = (acc_ref[pl.ds(0, _L)]
                                         + jnp.where(k > t, 1, 0))
                return carry

            jax.lax.fori_loop(0, _NCH, cbody, 0)
            return _xsum(red_ref, acc_ref[pl.ds(0, _L)])

        ls_ref[pl.ds(0, _L)] = jnp.full((_L,), _IMIN, jnp.int32)
        ls_ref[pl.ds(_L, _L)] = jnp.full((_L,), _IMAX, jnp.int32)

        def bs_body(i, carry):
            lo = ls_ref[pl.ds(0, _L)]
            hi = ls_ref[pl.ds(_L, _L)]
            mid = (lo & hi) + ((lo ^ hi) >> 1)
            ok = count(mid) >= kk
            ls_ref[pl.ds(0, _L)] = jnp.where(ok, mid, lo)
            ls_ref[pl.ds(_L, _L)] = jnp.where(ok, hi, mid)
            return carry

        jax.lax.fori_loop(0, 33, bs_body, 0)
        tstar = ls_ref[pl.ds(_L, _L)]
        need = kk - count(tstar)

        def gfun(iv):
            acc_ref[pl.ds(0, _L)] = jnp.zeros((_L,), jnp.int32)

            def gb(i, carry):
                k = keys_ref[pl.ds(i * _L, _L)]
                hit = (k == tstar) & ((lane + i * _L) < iv)
                acc_ref[pl.ds(0, _L)] = (acc_ref[pl.ds(0, _L)]
                                         + jnp.where(hit, 1, 0))
                return carry

            jax.lax.fori_loop(0, _NCH, gb, 0)
            return (_xsum(red_ref, acc_ref[pl.ds(0, _L)])
                    + jnp.where(iv > _NSC, 1, 0))

        ls_ref[pl.ds(0, _L)] = jnp.zeros((_L,), jnp.int32)
        ls_ref[pl.ds(_L, _L)] = jnp.full((_L,), _NSC + 1, jnp.int32)

        def ibs_body(i, carry):
            lo = ls_ref[pl.ds(0, _L)]
            hi = ls_ref[pl.ds(_L, _L)]
            mid = (lo + hi) >> 1
            ok = gfun(mid) <= need
            ls_ref[pl.ds(0, _L)] = jnp.where(ok, mid, lo)
            ls_ref[pl.ds(_L, _L)] = jnp.where(ok, hi, mid)
            return carry

        jax.lax.fori_loop(0, 13, ibs_body, 0)
        jstar = ls_ref[pl.ds(0, _L)]

        def mbody(i, carry):
            k = keys_ref[pl.ds(i * _L, _L)]
            keep = (k > tstar) | ((k == tstar) & ((lane + i * _L) < jstar))
            cm_ref[pl.ds(i * _L, _L)] = jnp.where(keep, jnp.float32(1.0),
                                                  jnp.float32(0.0))
            return carry

        jax.lax.fori_loop(0, _NCH, mbody, 0)
        cp3 = pltpu.make_async_copy(cm_ref, out_ref.at[b], sem)
        cp3.start()
        cp3.wait()


def _out_kernel(nsrc_ref, a1r_ref, a2r_ref, cmr_ref, a1c_ref, a2c_ref,
                cmc_ref, out_ref):
    b = pl.program_id(0)
    ti = pl.program_id(1)
    n = nsrc_ref[b]
    scale = jnp.float32(5.0) / (2 * n).astype(jnp.float32)
    ai = a1r_ref[0]
    a2i = a2r_ref[0]
    cmi = cmr_ref[0]
    aj = a1c_ref[0]
    a2j = a2c_ref[0]
    cmj = cmc_ref[0]
    tm = ai.shape[0]
    tn = aj.shape[-1]
    rid = ti * tm + jax.lax.broadcasted_iota(jnp.int32, (tm, 1), 0)
    cid = jax.lax.broadcasted_iota(jnp.int32, (1, tn), 1)
    v = jnp.maximum((ai + a2j) * jnp.float32(1.0 / _N_HEAD), 0.0)
    v = jnp.where((rid < n) & (cmj > 0), v, 0.0)
    gate = ((aj + a2i) > 0) & (cid < n) & (cmi > 0)
    out_ref[0] = scale * jnp.tanh(jnp.where(gate, v, 0.0))


def kernel(emb1, n_src, ns_tgt, W, a1, a2):
    B, N, IN_F = emb1.shape
    OUT_F = W.shape[0]
    ac = jnp.concatenate([a1, a2], axis=1).astype(jnp.bfloat16)
    wb = W.astype(jnp.bfloat16)

    BM = 512
    a12 = pl.pallas_call(
        _proj_kernel,
        grid=(B, N // BM),
        in_specs=[
            pl.BlockSpec((OUT_F, IN_F), lambda b, i: (0, 0)),
            pl.BlockSpec((OUT_F, 2), lambda b, i: (0, 0)),
            pl.BlockSpec((1, BM, IN_F), lambda b, i: (b, i, 0)),
        ],
        out_specs=pl.BlockSpec((1, BM, 2), lambda b, i: (b, i, 0)),
        out_shape=jax.ShapeDtypeStruct((B, N, 2), jnp.float32),
    )(wb, ac, emb1)

    a12_c = jnp.transpose(a12, (0, 2, 1))
    a1r = a12[:, :, 0:1]
    a2r = a12[:, :, 1:2]
    a1c = a12_c[:, 0:1, :]
    a2c = a12_c[:, 1:2, :]

    rank_fn = pl.kernel(
        _rank_sc_body,
        out_type=jax.ShapeDtypeStruct((B, N), jnp.float32),
        mesh=plsc.VectorSubcoreMesh(core_axis_name="c", subcore_axis_name="s"),
        scratch_types=[
            pltpu.VMEM((N,), jnp.float32),
            pltpu.VMEM((N,), jnp.int32),
            pltpu.VMEM((N,), jnp.float32),
            pltpu.VMEM((2 * _L,), jnp.int32),
            pltpu.VMEM((_L,), jnp.int32),
            pltpu.VMEM((_L,), jnp.int32),
            pltpu.VMEM((2 * _L,), jnp.int32),
            pltpu.SemaphoreType.DMA,
        ],
    )
    scal = jnp.concatenate(
        [n_src.astype(jnp.int32), ns_tgt.astype(jnp.int32),
         jnp.zeros(_L - 2 * B, jnp.int32)])
    cm2d = rank_fn(scal, a12_c[:, 1, :])
    cmc = cm2d[:, None, :]
    cmr = cm2d[:, :, None]

    TM = 256
    vec_c = pl.BlockSpec((1, 1, N), lambda b, i: (b, 0, 0))
    vec_r = pl.BlockSpec((1, TM, 1), lambda b, i: (b, i, 0))
    out = pl.pallas_call(
        _out_kernel,
        grid=(B, N // TM),
        in_specs=[
            pl.BlockSpec(memory_space=pltpu.SMEM),
            vec_r, vec_r, vec_r, vec_c, vec_c, vec_c,
        ],
        out_specs=pl.BlockSpec((1, TM, N), lambda b, i: (b, i, 0)),
        out_shape=jax.ShapeDtypeStruct((B, N, N), jnp.float32),
    )(n_src, a1r, a2r, cmr, a1c, a2c, cmc)
    return out


# speedup vs baseline: 1.0904x; 1.0904x over previous
import jax
import jax.numpy as jnp
import numpy as np
from jax.experimental import pallas as pl
from jax.experimental.pallas import tpu as pltpu
import jax.experimental.pallas.tpu_sc as plsc

_N_HEAD = 16
_L = 16
_IMIN = np.int32(-(2 ** 31))
_IMAX = np.int32(2 ** 31 - 1)
_NCH = 2048 // _L
_NSC = 2048


def _proj_kernel(w_ref, ac_ref, emb_ref, a12_ref):
    h = jax.lax.dot_general(emb_ref[0].astype(jnp.bfloat16), w_ref[...],
                            (((1,), (1,)), ((), ())),
                            preferred_element_type=jnp.float32)
    a12_ref[0] = jax.lax.dot_general(h.astype(jnp.bfloat16), ac_ref[...],
                                     (((1,), (0,)), ((), ())),
                                     preferred_element_type=jnp.float32)


def _xsum(red_ref, v):
    for sh in (1, 2, 4, 8):
        red_ref[pl.ds(0, _L)] = v
        red_ref[pl.ds(_L, _L)] = v
        v = v + red_ref[pl.ds(sh, _L)]
    return v


def _xmax(red_ref, v):
    for sh in (1, 2, 4, 8):
        red_ref[pl.ds(0, _L)] = v
        red_ref[pl.ds(_L, _L)] = v
        v = jnp.maximum(v, red_ref[pl.ds(sh, _L)])
    return v


def _rank_sc_body(scal_ref, a2v_ref, out_ref, buf_ref, keys_ref, cm_ref,
                  red_ref, sv_ref, acc_ref, ls_ref, sem):
    c = jax.lax.axis_index("c")
    s = jax.lax.axis_index("s")
    b = c * 16 + s

    batches = a2v_ref.shape[0]
    @pl.when(b < batches)
    def _():
        cp = pltpu.make_async_copy(scal_ref, sv_ref, sem)
        cp.start()
        cp.wait()
        cp2 = pltpu.make_async_copy(a2v_ref.at[b], buf_ref, sem)
        cp2.start()
        cp2.wait()
        lane = jax.lax.iota(jnp.int32, _L)
        sv = sv_ref[pl.ds(0, _L)]
        ns = _xmax(red_ref, jnp.where(lane == b, sv, _IMIN))
        nt = _xmax(red_ref, jnp.where(lane == b + 4, sv, _IMIN))
        kk = (2 * ns * 52429) >> 18

        def key_body(i, carry):
            x = buf_ref[pl.ds(i * _L, _L)]
            bits = jax.lax.bitcast_convert_type(x, jnp.int32)
            keym = bits ^ ((bits >> 31) & jnp.int32(0x7FFFFFFF))
            idx = lane + i * _L
            keys_ref[pl.ds(i * _L, _L)] = jnp.where(idx < nt, keym, _IMIN)
            return carry

        jax.lax.fori_loop(0, _NCH, key_body, 0)

        def count(t):
            acc_ref[pl.ds(0, _L)] = jnp.zeros((_L,), jnp.int32)

            def cbody(i, carry):
                a = jnp.zeros((_L,), jnp.int32)
                for u in range(4):
                    k = keys_ref[pl.ds((4 * i + u) * _L, _L)]
                    a = a + jnp.where(k > t, 1, 0)
                acc_ref[pl.ds(0, _L)] = acc_ref[pl.ds(0, _L)] + a
                return carry

            jax.lax.fori_loop(0, _NCH // 4, cbody, 0)
            return _xsum(red_ref, acc_ref[pl.ds(0, _L)])

        ls_ref[pl.ds(0, _L)] = jnp.full((_L,), _IMIN, jnp.int32)
        ls_ref[pl.ds(_L, _L)] = jnp.full((_L,), _IMAX, jnp.int32)

        def bs_body(i, carry):
            lo = ls_ref[pl.ds(0, _L)]
            hi = ls_ref[pl.ds(_L, _L)]
            mid = (lo & hi) + ((lo ^ hi) >> 1)
            ok = count(mid) >= kk
            ls_ref[pl.ds(0, _L)] = jnp.where(ok, mid, lo)
            ls_ref[pl.ds(_L, _L)] = jnp.where(ok, hi, mid)
            return carry

        jax.lax.fori_loop(0, 33, bs_body, 0)
        tstar = ls_ref[pl.ds(_L, _L)]
        need = kk - count(tstar)

        def gfun(iv):
            acc_ref[pl.ds(0, _L)] = jnp.zeros((_L,), jnp.int32)

            def gb(i, carry):
                k = keys_ref[pl.ds(i * _L, _L)]
                hit = (k == tstar) & ((lane + i * _L) < iv)
                acc_ref[pl.ds(0, _L)] = (acc_ref[pl.ds(0, _L)]
                                         + jnp.where(hit, 1, 0))
                return carry

            jax.lax.fori_loop(0, _NCH, gb, 0)
            return (_xsum(red_ref, acc_ref[pl.ds(0, _L)])
                    + jnp.where(iv > _NSC, 1, 0))

        ls_ref[pl.ds(0, _L)] = jnp.zeros((_L,), jnp.int32)
        ls_ref[pl.ds(_L, _L)] = jnp.full((_L,), _NSC + 1, jnp.int32)

        def ibs_body(i, carry):
            lo = ls_ref[pl.ds(0, _L)]
            hi = ls_ref[pl.ds(_L, _L)]
            mid = (lo + hi) >> 1
            ok = gfun(mid) <= need
            ls_ref[pl.ds(0, _L)] = jnp.where(ok, mid, lo)
            ls_ref[pl.ds(_L, _L)] = jnp.where(ok, hi, mid)
            return carry

        jax.lax.fori_loop(0, 13, ibs_body, 0)
        jstar = ls_ref[pl.ds(0, _L)]

        def mbody(i, carry):
            k = keys_ref[pl.ds(i * _L, _L)]
            keep = (k > tstar) | ((k == tstar) & ((lane + i * _L) < jstar))
            cm_ref[pl.ds(i * _L, _L)] = jnp.where(keep, jnp.float32(1.0),
                                                  jnp.float32(0.0))
            return carry

        jax.lax.fori_loop(0, _NCH, mbody, 0)
        cp3 = pltpu.make_async_copy(cm_ref, out_ref.at[b], sem)
        cp3.start()
        cp3.wait()


def _out_kernel(nsrc_ref, a1r_ref, a2r_ref, cmr_ref, a1c_ref, a2c_ref,
                cmc_ref, out_ref):
    b = pl.program_id(0)
    ti = pl.program_id(1)
    n = nsrc_ref[b]
    scale = jnp.float32(5.0) / (2 * n).astype(jnp.float32)
    ai = a1r_ref[0]
    a2i = a2r_ref[0]
    cmi = cmr_ref[0]
    aj = a1c_ref[0]
    a2j = a2c_ref[0]
    cmj = cmc_ref[0]
    tm = ai.shape[0]
    tn = aj.shape[-1]
    rid = ti * tm + jax.lax.broadcasted_iota(jnp.int32, (tm, 1), 0)
    cid = jax.lax.broadcasted_iota(jnp.int32, (1, tn), 1)
    v = jnp.maximum((ai + a2j) * jnp.float32(1.0 / _N_HEAD), 0.0)
    v = jnp.where((rid < n) & (cmj > 0), v, 0.0)
    gate = ((aj + a2i) > 0) & (cid < n) & (cmi > 0)
    out_ref[0] = scale * jnp.tanh(jnp.where(gate, v, 0.0))


def kernel(emb1, n_src, ns_tgt, W, a1, a2):
    B, N, IN_F = emb1.shape
    OUT_F = W.shape[0]
    ac = jnp.concatenate([a1, a2], axis=1).astype(jnp.bfloat16)
    wb = W.astype(jnp.bfloat16)

    BM = 512
    a12 = pl.pallas_call(
        _proj_kernel,
        grid=(B, N // BM),
        in_specs=[
            pl.BlockSpec((OUT_F, IN_F), lambda b, i: (0, 0)),
            pl.BlockSpec((OUT_F, 2), lambda b, i: (0, 0)),
            pl.BlockSpec((1, BM, IN_F), lambda b, i: (b, i, 0)),
        ],
        out_specs=pl.BlockSpec((1, BM, 2), lambda b, i: (b, i, 0)),
        out_shape=jax.ShapeDtypeStruct((B, N, 2), jnp.float32),
    )(wb, ac, emb1)

    a12_c = jnp.transpose(a12, (0, 2, 1))
    a1r = a12[:, :, 0:1]
    a2r = a12[:, :, 1:2]
    a1c = a12_c[:, 0:1, :]
    a2c = a12_c[:, 1:2, :]

    rank_fn = pl.kernel(
        _rank_sc_body,
        out_type=jax.ShapeDtypeStruct((B, N), jnp.float32),
        mesh=plsc.VectorSubcoreMesh(core_axis_name="c", subcore_axis_name="s"),
        scratch_types=[
            pltpu.VMEM((N,), jnp.float32),
            pltpu.VMEM((N,), jnp.int32),
            pltpu.VMEM((N,), jnp.float32),
            pltpu.VMEM((2 * _L,), jnp.int32),
            pltpu.VMEM((_L,), jnp.int32),
            pltpu.VMEM((_L,), jnp.int32),
            pltpu.VMEM((2 * _L,), jnp.int32),
            pltpu.SemaphoreType.DMA,
        ],
    )
    scal = jnp.concatenate(
        [n_src.astype(jnp.int32), ns_tgt.astype(jnp.int32),
         jnp.zeros(_L - 2 * B, jnp.int32)])
    cm2d = rank_fn(scal, a12_c[:, 1, :])
    cmc = cm2d[:, None, :]
    cmr = cm2d[:, :, None]

    TM = 256
    vec_c = pl.BlockSpec((1, 1, N), lambda b, i: (b, 0, 0))
    vec_r = pl.BlockSpec((1, TM, 1), lambda b, i: (b, i, 0))
    out = pl.pallas_call(
        _out_kernel,
        grid=(B, N // TM),
        in_specs=[
            pl.BlockSpec(memory_space=pltpu.SMEM),
            vec_r, vec_r, vec_r, vec_c, vec_c, vec_c,
        ],
        out_specs=pl.BlockSpec((1, TM, N), lambda b, i: (b, i, 0)),
        out_shape=jax.ShapeDtypeStruct((B, N, N), jnp.float32),
    )(n_src, a1r, a2r, cmr, a1c, a2c, cmc)
    return out
